# baseline (speedup 1.0000x reference)
import functools

import jax
import jax.numpy as jnp
from jax import lax
from jax.experimental import pallas as pl
from jax.experimental.pallas import tpu as pltpu
from jax.experimental.pallas import tpu_sc as plsc

H = 768
I = 128
E = 16
SH_I = 256
T = 2048
TB = 128
NB = T // TB
BG = 128
MAXB = 31
XS = 4096
NT = 16
TPT = T // NT


def _dot_nt(a, b):
    return jax.lax.dot_general(a, b, (((1,), (1,)), ((), ())),
                               preferred_element_type=jnp.float32)


def _gate_block(x_ref, gw_ref, eid_ref, wt_ref):
    xb = x_ref[...]
    scores = jax.lax.dot_general(xb.astype(jnp.bfloat16),
                                 gw_ref[...].astype(jnp.bfloat16),
                                 (((1,), (1,)), ((), ())),
                                 preferred_element_type=jnp.float32)
    smax = jnp.max(scores, axis=-1, keepdims=True)
    p = jnp.exp(scores - smax)
    probs = p / jnp.sum(p, axis=-1, keepdims=True)
    pmax = jnp.max(probs, axis=-1, keepdims=True)
    lane = jax.lax.broadcasted_iota(jnp.int32, (TB, E), 1)
    fm = jnp.min(jnp.where(probs >= pmax, lane, E), axis=-1,
                 keepdims=True)
    eid_ref[...] = fm
    wt_ref[...] = pmax


def _gate(x, gate_w):
    return pl.pallas_call(
        _gate_block,
        grid=(NB,),
        in_specs=[pl.BlockSpec((TB, H), lambda b: (b, 0)),
                  pl.BlockSpec((E, H), lambda b: (0, 0))],
        out_specs=[pl.BlockSpec((TB, 1), lambda b: (b, 0)),
                   pl.BlockSpec((TB, 1), lambda b: (b, 0))],
        out_shape=[jax.ShapeDtypeStruct((T, 1), jnp.int32),
                   jax.ShapeDtypeStruct((T, 1), jnp.float32)],
    )(x, gate_w)


def _lane(vec, i):
    iota = lax.iota(jnp.int32, 16)
    return jnp.sum(jnp.where(iota == i, vec, 0))


def _route_body(eid_hbm, x_hbm, xs_hbm, pos_hbm, gid_hbm, nblk_hbm,
                eid_v, pos_v, rows_v, cnt_v, cnt_all_v, gid_v, nblk_v,
                cnt_sh, sem):
    sid = lax.axis_index("s")
    base = sid * TPT
    iota = lax.iota(jnp.int32, 16)

    pltpu.sync_copy(eid_hbm.at[pl.ds(base, TPT)], eid_v)

    cnt = jnp.zeros((16,), jnp.int32)
    chunks = []
    for v in range(TPT // 16):
        ev = eid_v[pl.ds(v * 16, 16)]
        chunks.append(ev)
        for e in range(E):
            c = jnp.sum(jnp.where(ev == e, 1, 0))
            cnt = cnt + jnp.where(iota == e, c, 0)
    cnt_v[...] = cnt

    pltpu.sync_copy(cnt_v, cnt_sh.at[pl.ds(sid * 16, 16)])
    plsc.subcore_barrier()
    pltpu.sync_copy(cnt_sh, cnt_all_v)

    total = jnp.zeros((16,), jnp.int32)
    for w in range(NT):
        total = total + cnt_all_v[pl.ds(w * 16, 16)]
    ptot = jnp.bitwise_and(total + (BG - 1), -BG)
    csum = plsc.cumsum(ptot)
    pbase = csum - ptot

    tbase = pbase
    for w in range(NT):
        before = jnp.full((16,), w, jnp.int32) < sid
        tbase = tbase + jnp.where(before, cnt_all_v[pl.ds(w * 16, 16)], 0)

    ctr = tbase
    for v in range(TPT // 16):
        ev = chunks[v]
        posk = jnp.zeros((16,), jnp.int32)
        for e in range(E):
            m = ev == e
            mi = jnp.where(m, 1, 0)
            r = plsc.cumsum(mi)
            basee = _lane(ctr, e)
            posk = jnp.where(m, basee + r - 1, posk)
            ctr = ctr + jnp.where(iota == e, _lane(r, 15), 0)
        pos_v[pl.ds(v * 16, 16)] = posk

    pltpu.sync_copy(pos_v, pos_hbm.at[pl.ds(base, TPT)])

    pltpu.sync_copy(x_hbm.at[pl.ds(base, TPT)], rows_v)
    pltpu.async_copy(rows_v, xs_hbm.at[pos_v], sem).wait()

    @pl.when(sid == 0)
    def _():
        nblk_v[...] = (_lane(csum, 15) >> 7) + jnp.zeros((16,), jnp.int32)
        bstart = pbase >> 7
        for v in range(2):
            pvec = iota + v * 16
            cntb = jnp.zeros((16,), jnp.int32)
            for e in range(E):
                bs = _lane(bstart, e)
                cntb = cntb + jnp.where(bs <= pvec, 1, 0)
            gid_v[pl.ds(v * 16, 16)] = cntb - 1
        pltpu.sync_copy(gid_v, gid_hbm)
        pltpu.sync_copy(nblk_v, nblk_hbm)


def _route(eid, x):
    mesh = plsc.VectorSubcoreMesh(core_axis_name="c", subcore_axis_name="s",
                                  num_cores=1)
    return pl.kernel(
        _route_body,
        out_type=(jax.ShapeDtypeStruct((XS, H), jnp.float32),
                  jax.ShapeDtypeStruct((T,), jnp.int32),
                  jax.ShapeDtypeStruct((32,), jnp.int32),
                  jax.ShapeDtypeStruct((16,), jnp.int32)),
        mesh=mesh,
        scratch_types=[
            pltpu.VMEM((TPT,), jnp.int32),
            pltpu.VMEM((TPT,), jnp.int32),
            pltpu.VMEM((TPT, H), jnp.float32),
            pltpu.VMEM((16,), jnp.int32),
            pltpu.VMEM((NT * 16,), jnp.int32),
            pltpu.VMEM((32,), jnp.int32),
            pltpu.VMEM((16,), jnp.int32),
            pltpu.VMEM_SHARED((NT * 16,), jnp.int32),
            pltpu.SemaphoreType.DMA,
        ],
        compiler_params=pltpu.CompilerParams(needs_layout_passes=False),
    )(eid, x)


def _shared_block(x_ref, sw1_ref, sb1_ref, sw2_ref, sb2_ref, sw3_ref,
                  sb3_ref, z_ref):
    xb = x_ref[...]
    t1 = _dot_nt(xb, sw1_ref[...]) + sb1_ref[...]
    t3 = _dot_nt(xb, sw3_ref[...]) + sb3_ref[...]
    z_ref[...] = _dot_nt(jax.nn.silu(t1) * t3, sw2_ref[...]) + sb2_ref[...]


def _shared(x, sw1, sb1, sw2, sb2, sw3, sb3):
    full = lambda shape: pl.BlockSpec(shape, lambda b: (0,) * len(shape))
    return pl.pallas_call(
        _shared_block,
        grid=(NB,),
        in_specs=[pl.BlockSpec((TB, H), lambda b: (b, 0)),
                  full((SH_I, H)), full((1, SH_I)),
                  full((H, SH_I)), full((1, H)),
                  full((SH_I, H)), full((1, SH_I))],
        out_specs=pl.BlockSpec((TB, H), lambda b: (b, 0)),
        out_shape=jax.ShapeDtypeStruct((T, H), jnp.float32),
    )(x, sw1, sb1.reshape(1, SH_I), sw2, sb2.reshape(1, H), sw3,
      sb3.reshape(1, SH_I))


def _ffn_block(gid_ref, nblk_ref, xs_ref, ew1_ref, eb1_ref, ew2_ref,
               eb2_ref, ew3_ref, eb3_ref, ys_ref):
    g = pl.program_id(0)

    @pl.when(g < nblk_ref[0])
    def _():
        xb = xs_ref[...]
        h1 = _dot_nt(xb, ew1_ref[0]) + eb1_ref[0]
        h3 = _dot_nt(xb, ew3_ref[0]) + eb3_ref[0]
        h = jax.nn.silu(h1) * h3
        ys_ref[...] = _dot_nt(h, ew2_ref[0]) + eb2_ref[0]


def _ffn(gid, nblk, xs, ew1, eb1, ew2, eb2, ew3, eb3):
    grid_spec = pltpu.PrefetchScalarGridSpec(
        num_scalar_prefetch=2,
        grid=(MAXB,),
        in_specs=[
            pl.BlockSpec((BG, H), lambda g, gid_ref, nblk_ref: (g, 0)),
            pl.BlockSpec((1, I, H),
                         lambda g, gid_ref, nblk_ref: (gid_ref[g], 0, 0)),
            pl.BlockSpec((1, 1, I),
                         lambda g, gid_ref, nblk_ref: (gid_ref[g], 0, 0)),
            pl.BlockSpec((1, H, I),
                         lambda g, gid_ref, nblk_ref: (gid_ref[g], 0, 0)),
            pl.BlockSpec((1, 1, H),
                         lambda g, gid_ref, nblk_ref: (gid_ref[g], 0, 0)),
            pl.BlockSpec((1, I, H),
                         lambda g, gid_ref, nblk_ref: (gid_ref[g], 0, 0)),
            pl.BlockSpec((1, 1, I),
                         lambda g, gid_ref, nblk_ref: (gid_ref[g], 0, 0)),
        ],
        out_specs=pl.BlockSpec((BG, H), lambda g, gid_ref, nblk_ref: (g, 0)),
    )
    return pl.pallas_call(
        _ffn_block,
        grid_spec=grid_spec,
        out_shape=jax.ShapeDtypeStruct((XS, H), jnp.float32),
    )(gid, nblk, xs, ew1, eb1.reshape(E, 1, I), ew2, eb2.reshape(E, 1, H),
      ew3, eb3.reshape(E, 1, I))


def _combine_body(pos_hbm, ys_hbm, yr_hbm, idx_v, rows_v, sem):
    info = plsc.get_sparse_core_info()
    nc = info.num_cores
    wid = lax.axis_index("s") * nc + lax.axis_index("c")
    n = T // (nc * info.num_subcores)
    base = wid * n
    pltpu.sync_copy(pos_hbm.at[pl.ds(base, n)], idx_v)
    pltpu.async_copy(ys_hbm.at[idx_v], rows_v, sem).wait()
    pltpu.sync_copy(rows_v, yr_hbm.at[pl.ds(base, n)])


def _combine(pos, ys):
    info = plsc.get_sparse_core_info()
    n = T // (info.num_cores * info.num_subcores)
    mesh = plsc.VectorSubcoreMesh(core_axis_name="c", subcore_axis_name="s")
    return pl.kernel(
        _combine_body,
        out_type=jax.ShapeDtypeStruct((T, H), jnp.float32),
        mesh=mesh,
        scratch_types=[
            pltpu.VMEM((n,), jnp.int32),
            pltpu.VMEM((n, H), jnp.float32),
            pltpu.SemaphoreType.DMA,
        ],
        compiler_params=pltpu.CompilerParams(needs_layout_passes=False),
    )(pos, ys)


def _epi_block(yr_ref, wt_ref, z_ref, out_ref):
    out_ref[...] = yr_ref[...] * wt_ref[...] + z_ref[...]


def _epilogue(yr, wt, z):
    return pl.pallas_call(
        _epi_block,
        grid=(NB,),
        in_specs=[pl.BlockSpec((TB, H), lambda b: (b, 0)),
                  pl.BlockSpec((TB, 1), lambda b: (b, 0)),
                  pl.BlockSpec((TB, H), lambda b: (b, 0))],
        out_specs=pl.BlockSpec((TB, H), lambda b: (b, 0)),
        out_shape=jax.ShapeDtypeStruct((T, H), jnp.float32),
    )(yr, wt, z)


@jax.jit
def _moe(x, gate_w, ew1, eb1, ew2, eb2, ew3, eb3, sw1, sb1, sw2, sb2,
         sw3, sb3):
    eid2d, wt2d = _gate(x, gate_w)
    eid = eid2d.reshape(T)
    xs, pos, gid, nblk = _route(eid, x)
    z = _shared(x, sw1, sb1, sw2, sb2, sw3, sb3)
    ys = _ffn(gid, nblk, xs, ew1, eb1, ew2, eb2, ew3, eb3)
    yr = _combine(pos, ys)
    return _epilogue(yr, wt2d, z)


def kernel(hidden_states, gate_w, ew1, eb1, ew2, eb2, ew3, eb3, sw1, sb1,
           sw2, sb2, sw3, sb3):
    shape = hidden_states.shape
    x = hidden_states.reshape(-1, H)
    y = _moe(x, gate_w, ew1, eb1, ew2, eb2, ew3, eb3, sw1, sb1, sw2, sb2,
             sw3, sb3)
    return y.reshape(shape)


# speedup vs baseline: 1.0229x; 1.0229x over previous
import functools

import jax
import jax.numpy as jnp
from jax import lax
from jax.experimental import pallas as pl
from jax.experimental.pallas import tpu as pltpu
from jax.experimental.pallas import tpu_sc as plsc

H = 768
I = 128
E = 16
SH_I = 256
T = 2048
TB = 128
NB = T // TB
BG = 128
MAXB = 31
XS = 4096
NT = 16
TPT = T // NT


def _dot_nt(a, b):
    return jax.lax.dot_general(a, b, (((1,), (1,)), ((), ())),
                               preferred_element_type=jnp.float32)


def _gate_block(x_ref, gw_ref, eid_ref, wt_ref):
    xb = x_ref[...]
    scores = jax.lax.dot_general(xb.astype(jnp.bfloat16),
                                 gw_ref[...].astype(jnp.bfloat16),
                                 (((1,), (1,)), ((), ())),
                                 preferred_element_type=jnp.float32)
    smax = jnp.max(scores, axis=-1, keepdims=True)
    p = jnp.exp(scores - smax)
    probs = p / jnp.sum(p, axis=-1, keepdims=True)
    pmax = jnp.max(probs, axis=-1, keepdims=True)
    lane = jax.lax.broadcasted_iota(jnp.int32, (TB, E), 1)
    fm = jnp.min(jnp.where(probs >= pmax, lane, E), axis=-1,
                 keepdims=True)
    eid_ref[...] = fm
    wt_ref[...] = pmax


def _gate(x, gate_w):
    return pl.pallas_call(
        _gate_block,
        grid=(NB,),
        in_specs=[pl.BlockSpec((TB, H), lambda b: (b, 0)),
                  pl.BlockSpec((E, H), lambda b: (0, 0))],
        out_specs=[pl.BlockSpec((TB, 1), lambda b: (b, 0)),
                   pl.BlockSpec((TB, 1), lambda b: (b, 0))],
        out_shape=[jax.ShapeDtypeStruct((T, 1), jnp.int32),
                   jax.ShapeDtypeStruct((T, 1), jnp.float32)],
    )(x, gate_w)


def _lane(vec, i):
    iota = lax.iota(jnp.int32, 16)
    return jnp.sum(jnp.where(iota == i, vec, 0))


def _route_body(eid_hbm, x_hbm, xs_hbm, pos_hbm, gid_hbm, nblk_hbm,
                eid_v, pos_v, rows_v, cnt_v, cnt_all_v, gid_v, nblk_v,
                cnt_sh, sem):
    sid = lax.axis_index("s")
    base = sid * TPT
    iota = lax.iota(jnp.int32, 16)

    rows_dma = pltpu.async_copy(x_hbm.at[pl.ds(base, TPT)], rows_v, sem)
    pltpu.sync_copy(eid_hbm.at[pl.ds(base, TPT)], eid_v)

    cnt = jnp.zeros((16,), jnp.int32)
    chunks = []
    for v in range(TPT // 16):
        ev = eid_v[pl.ds(v * 16, 16)]
        chunks.append(ev)
        for e in range(E):
            c = jnp.sum(jnp.where(ev == e, 1, 0))
            cnt = cnt + jnp.where(iota == e, c, 0)
    cnt_v[...] = cnt

    pltpu.sync_copy(cnt_v, cnt_sh.at[pl.ds(sid * 16, 16)])
    plsc.subcore_barrier()
    pltpu.sync_copy(cnt_sh, cnt_all_v)

    total = jnp.zeros((16,), jnp.int32)
    for w in range(NT):
        total = total + cnt_all_v[pl.ds(w * 16, 16)]
    ptot = jnp.bitwise_and(total + (BG - 1), -BG)
    csum = plsc.cumsum(ptot)
    pbase = csum - ptot

    tbase = pbase
    for w in range(NT):
        before = jnp.full((16,), w, jnp.int32) < sid
        tbase = tbase + jnp.where(before, cnt_all_v[pl.ds(w * 16, 16)], 0)

    ctr = tbase
    for v in range(TPT // 16):
        ev = chunks[v]
        posk = jnp.zeros((16,), jnp.int32)
        for e in range(E):
            m = ev == e
            mi = jnp.where(m, 1, 0)
            r = plsc.cumsum(mi)
            basee = _lane(ctr, e)
            posk = jnp.where(m, basee + r - 1, posk)
            ctr = ctr + jnp.where(iota == e, _lane(r, 15), 0)
        pos_v[pl.ds(v * 16, 16)] = posk

    pltpu.sync_copy(pos_v, pos_hbm.at[pl.ds(base, TPT)])

    rows_dma.wait()
    pltpu.async_copy(rows_v, xs_hbm.at[pos_v], sem).wait()

    @pl.when(sid == 0)
    def _():
        nblk_v[...] = (_lane(csum, 15) >> 7) + jnp.zeros((16,), jnp.int32)
        bstart = pbase >> 7
        for v in range(2):
            pvec = iota + v * 16
            cntb = jnp.zeros((16,), jnp.int32)
            for e in range(E):
                bs = _lane(bstart, e)
                cntb = cntb + jnp.where(bs <= pvec, 1, 0)
            gid_v[pl.ds(v * 16, 16)] = cntb - 1
        pltpu.sync_copy(gid_v, gid_hbm)
        pltpu.sync_copy(nblk_v, nblk_hbm)


def _route(eid, x):
    mesh = plsc.VectorSubcoreMesh(core_axis_name="c", subcore_axis_name="s",
                                  num_cores=1)
    return pl.kernel(
        _route_body,
        out_type=(jax.ShapeDtypeStruct((XS, H), jnp.float32),
                  jax.ShapeDtypeStruct((T,), jnp.int32),
                  jax.ShapeDtypeStruct((32,), jnp.int32),
                  jax.ShapeDtypeStruct((16,), jnp.int32)),
        mesh=mesh,
        scratch_types=[
            pltpu.VMEM((TPT,), jnp.int32),
            pltpu.VMEM((TPT,), jnp.int32),
            pltpu.VMEM((TPT, H), jnp.float32),
            pltpu.VMEM((16,), jnp.int32),
            pltpu.VMEM((NT * 16,), jnp.int32),
            pltpu.VMEM((32,), jnp.int32),
            pltpu.VMEM((16,), jnp.int32),
            pltpu.VMEM_SHARED((NT * 16,), jnp.int32),
            pltpu.SemaphoreType.DMA,
        ],
        compiler_params=pltpu.CompilerParams(needs_layout_passes=False),
    )(eid, x)


def _shared_block(x_ref, sw1_ref, sb1_ref, sw2_ref, sb2_ref, sw3_ref,
                  sb3_ref, z_ref):
    xb = x_ref[...]
    t1 = _dot_nt(xb, sw1_ref[...]) + sb1_ref[...]
    t3 = _dot_nt(xb, sw3_ref[...]) + sb3_ref[...]
    z_ref[...] = _dot_nt(jax.nn.silu(t1) * t3, sw2_ref[...]) + sb2_ref[...]


def _shared(x, sw1, sb1, sw2, sb2, sw3, sb3):
    full = lambda shape: pl.BlockSpec(shape, lambda b: (0,) * len(shape))
    return pl.pallas_call(
        _shared_block,
        grid=(NB,),
        in_specs=[pl.BlockSpec((TB, H), lambda b: (b, 0)),
                  full((SH_I, H)), full((1, SH_I)),
                  full((H, SH_I)), full((1, H)),
                  full((SH_I, H)), full((1, SH_I))],
        out_specs=pl.BlockSpec((TB, H), lambda b: (b, 0)),
        out_shape=jax.ShapeDtypeStruct((T, H), jnp.float32),
    )(x, sw1, sb1.reshape(1, SH_I), sw2, sb2.reshape(1, H), sw3,
      sb3.reshape(1, SH_I))


def _ffn_block(gid_ref, nblk_ref, xs_ref, ew1_ref, eb1_ref, ew2_ref,
               eb2_ref, ew3_ref, eb3_ref, ys_ref):
    g = pl.program_id(0)

    @pl.when(g < nblk_ref[0])
    def _():
        e = gid_ref[g]
        xb = xs_ref[...]
        h1 = _dot_nt(xb, ew1_ref[e]) + eb1_ref[e]
        h3 = _dot_nt(xb, ew3_ref[e]) + eb3_ref[e]
        h = jax.nn.silu(h1) * h3
        ys_ref[...] = _dot_nt(h, ew2_ref[e]) + eb2_ref[e]


def _ffn(gid, nblk, xs, ew1, eb1, ew2, eb2, ew3, eb3):
    full = lambda shape: pl.BlockSpec(
        shape, lambda g, gid_ref, nblk_ref: (0,) * len(shape))
    grid_spec = pltpu.PrefetchScalarGridSpec(
        num_scalar_prefetch=2,
        grid=(MAXB,),
        in_specs=[
            pl.BlockSpec((BG, H), lambda g, gid_ref, nblk_ref: (g, 0)),
            full((E, I, H)), full((E, 1, I)),
            full((E, H, I)), full((E, 1, H)),
            full((E, I, H)), full((E, 1, I)),
        ],
        out_specs=pl.BlockSpec((BG, H), lambda g, gid_ref, nblk_ref: (g, 0)),
    )
    return pl.pallas_call(
        _ffn_block,
        grid_spec=grid_spec,
        out_shape=jax.ShapeDtypeStruct((XS, H), jnp.float32),
    )(gid, nblk, xs, ew1, eb1.reshape(E, 1, I), ew2, eb2.reshape(E, 1, H),
      ew3, eb3.reshape(E, 1, I))


def _combine_body(pos_hbm, ys_hbm, yr_hbm, idx_v, rows_v, sem):
    info = plsc.get_sparse_core_info()
    nc = info.num_cores
    wid = lax.axis_index("s") * nc + lax.axis_index("c")
    n = T // (nc * info.num_subcores)
    base = wid * n
    pltpu.sync_copy(pos_hbm.at[pl.ds(base, n)], idx_v)
    pltpu.async_copy(ys_hbm.at[idx_v], rows_v, sem).wait()
    pltpu.sync_copy(rows_v, yr_hbm.at[pl.ds(base, n)])


def _combine(pos, ys):
    info = plsc.get_sparse_core_info()
    n = T // (info.num_cores * info.num_subcores)
    mesh = plsc.VectorSubcoreMesh(core_axis_name="c", subcore_axis_name="s")
    return pl.kernel(
        _combine_body,
        out_type=jax.ShapeDtypeStruct((T, H), jnp.float32),
        mesh=mesh,
        scratch_types=[
            pltpu.VMEM((n,), jnp.int32),
            pltpu.VMEM((n, H), jnp.float32),
            pltpu.SemaphoreType.DMA,
        ],
        compiler_params=pltpu.CompilerParams(needs_layout_passes=False),
    )(pos, ys)


def _epi_block(yr_ref, wt_ref, z_ref, out_ref):
    out_ref[...] = yr_ref[...] * wt_ref[...] + z_ref[...]


def _epilogue(yr, wt, z):
    return pl.pallas_call(
        _epi_block,
        grid=(NB,),
        in_specs=[pl.BlockSpec((TB, H), lambda b: (b, 0)),
                  pl.BlockSpec((TB, 1), lambda b: (b, 0)),
                  pl.BlockSpec((TB, H), lambda b: (b, 0))],
        out_specs=pl.BlockSpec((TB, H), lambda b: (b, 0)),
        out_shape=jax.ShapeDtypeStruct((T, H), jnp.float32),
    )(yr, wt, z)


@jax.jit
def _moe(x, gate_w, ew1, eb1, ew2, eb2, ew3, eb3, sw1, sb1, sw2, sb2,
         sw3, sb3):
    eid2d, wt2d = _gate(x, gate_w)
    eid = eid2d.reshape(T)
    xs, pos, gid, nblk = _route(eid, x)
    z = _shared(x, sw1, sb1, sw2, sb2, sw3, sb3)
    ys = _ffn(gid, nblk, xs, ew1, eb1, ew2, eb2, ew3, eb3)
    yr = _combine(pos, ys)
    return _epilogue(yr, wt2d, z)


def kernel(hidden_states, gate_w, ew1, eb1, ew2, eb2, ew3, eb3, sw1, sb1,
           sw2, sb2, sw3, sb3):
    shape = hidden_states.shape
    x = hidden_states.reshape(-1, H)
    y = _moe(x, gate_w, ew1, eb1, ew2, eb2, ew3, eb3, sw1, sb1, sw2, sb2,
             sw3, sb3)
    return y.reshape(shape)


# speedup vs baseline: 1.1150x; 1.0901x over previous
import functools

import jax
import jax.numpy as jnp
from jax import lax
from jax.experimental import pallas as pl
from jax.experimental.pallas import tpu as pltpu
from jax.experimental.pallas import tpu_sc as plsc

H = 768
I = 128
E = 16
SH_I = 256
T = 2048
TB = 128
NB = T // TB
BG = 128
MAXB = 31
XS = 4096
NT = 16
TPT = T // NT


def _dot_nt(a, b):
    return jax.lax.dot_general(a, b, (((1,), (1,)), ((), ())),
                               preferred_element_type=jnp.float32)


def _gate_block(x_ref, gw_ref, eid_ref, wt_ref):
    xb = x_ref[...]
    scores = jax.lax.dot_general(xb.astype(jnp.bfloat16),
                                 gw_ref[...].astype(jnp.bfloat16),
                                 (((1,), (1,)), ((), ())),
                                 preferred_element_type=jnp.float32)
    smax = jnp.max(scores, axis=-1, keepdims=True)
    p = jnp.exp(scores - smax)
    probs = p / jnp.sum(p, axis=-1, keepdims=True)
    pmax = jnp.max(probs, axis=-1, keepdims=True)
    lane = jax.lax.broadcasted_iota(jnp.int32, (TB, E), 1)
    fm = jnp.min(jnp.where(probs >= pmax, lane, E), axis=-1,
                 keepdims=True)
    eid_ref[...] = fm
    wt_ref[...] = pmax


def _gate(x, gate_w):
    return pl.pallas_call(
        _gate_block,
        grid=(NB,),
        in_specs=[pl.BlockSpec((TB, H), lambda b: (b, 0)),
                  pl.BlockSpec((E, H), lambda b: (0, 0))],
        out_specs=[pl.BlockSpec((TB, 1), lambda b: (b, 0)),
                   pl.BlockSpec((TB, 1), lambda b: (b, 0))],
        out_shape=[jax.ShapeDtypeStruct((T, 1), jnp.int32),
                   jax.ShapeDtypeStruct((T, 1), jnp.float32)],
    )(x, gate_w)


def _lane(vec, i):
    iota = lax.iota(jnp.int32, 16)
    return jnp.sum(jnp.where(iota == i, vec, 0))


def _route_body(eid_hbm, x_hbm, xs_hbm, pos_hbm, gid_hbm, nblk_hbm,
                eid_v, pos_v, rows_v, cnt_v, cnt_all_v, gid_v, nblk_v,
                cnt_sh, sem):
    sid = lax.axis_index("s")
    base = sid * TPT
    iota = lax.iota(jnp.int32, 16)

    rows_dma = pltpu.async_copy(x_hbm.at[pl.ds(base, TPT)], rows_v, sem)
    pltpu.sync_copy(eid_hbm.at[pl.ds(base, TPT)], eid_v)

    cnt = jnp.zeros((16,), jnp.int32)
    chunks = []
    for v in range(TPT // 16):
        ev = eid_v[pl.ds(v * 16, 16)]
        chunks.append(ev)
        for e in range(E):
            c = jnp.sum(jnp.where(ev == e, 1, 0))
            cnt = cnt + jnp.where(iota == e, c, 0)
    cnt_v[...] = cnt

    pltpu.sync_copy(cnt_v, cnt_sh.at[pl.ds(sid * 16, 16)])
    plsc.subcore_barrier()
    pltpu.sync_copy(cnt_sh, cnt_all_v)

    total = jnp.zeros((16,), jnp.int32)
    for w in range(NT):
        total = total + cnt_all_v[pl.ds(w * 16, 16)]
    ptot = jnp.bitwise_and(total + (BG - 1), -BG)
    csum = plsc.cumsum(ptot)
    pbase = csum - ptot

    tbase = pbase
    for w in range(NT):
        before = jnp.full((16,), w, jnp.int32) < sid
        tbase = tbase + jnp.where(before, cnt_all_v[pl.ds(w * 16, 16)], 0)

    ctr = tbase
    for v in range(TPT // 16):
        ev = chunks[v]
        posk = jnp.zeros((16,), jnp.int32)
        for e in range(E):
            m = ev == e
            mi = jnp.where(m, 1, 0)
            r = plsc.cumsum(mi)
            basee = _lane(ctr, e)
            posk = jnp.where(m, basee + r - 1, posk)
            ctr = ctr + jnp.where(iota == e, _lane(r, 15), 0)
        pos_v[pl.ds(v * 16, 16)] = posk

    pltpu.sync_copy(pos_v, pos_hbm.at[pl.ds(base, TPT)])

    rows_dma.wait()
    pltpu.async_copy(rows_v, xs_hbm.at[pos_v], sem).wait()

    @pl.when(sid == 0)
    def _():
        nblk_v[...] = (_lane(csum, 15) >> 7) + jnp.zeros((16,), jnp.int32)
        bstart = pbase >> 7
        for v in range(2):
            pvec = iota + v * 16
            cntb = jnp.zeros((16,), jnp.int32)
            for e in range(E):
                bs = _lane(bstart, e)
                cntb = cntb + jnp.where(bs <= pvec, 1, 0)
            gid_v[pl.ds(v * 16, 16)] = cntb - 1
        pltpu.sync_copy(gid_v, gid_hbm)
        pltpu.sync_copy(nblk_v, nblk_hbm)


def _route(eid, x):
    mesh = plsc.VectorSubcoreMesh(core_axis_name="c", subcore_axis_name="s",
                                  num_cores=1)
    return pl.kernel(
        _route_body,
        out_type=(jax.ShapeDtypeStruct((XS, H), jnp.float32),
                  jax.ShapeDtypeStruct((T,), jnp.int32),
                  jax.ShapeDtypeStruct((32,), jnp.int32),
                  jax.ShapeDtypeStruct((16,), jnp.int32)),
        mesh=mesh,
        scratch_types=[
            pltpu.VMEM((TPT,), jnp.int32),
            pltpu.VMEM((TPT,), jnp.int32),
            pltpu.VMEM((TPT, H), jnp.float32),
            pltpu.VMEM((16,), jnp.int32),
            pltpu.VMEM((NT * 16,), jnp.int32),
            pltpu.VMEM((32,), jnp.int32),
            pltpu.VMEM((16,), jnp.int32),
            pltpu.VMEM_SHARED((NT * 16,), jnp.int32),
            pltpu.SemaphoreType.DMA,
        ],
        compiler_params=pltpu.CompilerParams(needs_layout_passes=False),
    )(eid, x)


def _shared_block(x_ref, sw1_ref, sb1_ref, sw2_ref, sb2_ref, sw3_ref,
                  sb3_ref, z_ref):
    xb = x_ref[...]
    t1 = _dot_nt(xb, sw1_ref[...]) + sb1_ref[...]
    t3 = _dot_nt(xb, sw3_ref[...]) + sb3_ref[...]
    z_ref[...] = _dot_nt(jax.nn.silu(t1) * t3, sw2_ref[...]) + sb2_ref[...]


def _shared(x, sw1, sb1, sw2, sb2, sw3, sb3):
    full = lambda shape: pl.BlockSpec(shape, lambda b: (0,) * len(shape))
    return pl.pallas_call(
        _shared_block,
        grid=(NB,),
        in_specs=[pl.BlockSpec((TB, H), lambda b: (b, 0)),
                  full((SH_I, H)), full((1, SH_I)),
                  full((H, SH_I)), full((1, H)),
                  full((SH_I, H)), full((1, SH_I))],
        out_specs=pl.BlockSpec((TB, H), lambda b: (b, 0)),
        out_shape=jax.ShapeDtypeStruct((T, H), jnp.float32),
    )(x, sw1, sb1.reshape(1, SH_I), sw2, sb2.reshape(1, H), sw3,
      sb3.reshape(1, SH_I))


def _ffn_block(gid_ref, nblk_ref, xs_ref, ew1_ref, eb1_ref, ew2_ref,
               eb2_ref, ew3_ref, eb3_ref, ys_ref):
    g = pl.program_id(0)

    @pl.when(g < nblk_ref[0])
    def _():
        e = gid_ref[g]
        xb = xs_ref[...]
        h1 = _dot_nt(xb, ew1_ref[e]) + eb1_ref[e]
        h3 = _dot_nt(xb, ew3_ref[e]) + eb3_ref[e]
        h = jax.nn.silu(h1) * h3
        ys_ref[...] = _dot_nt(h, ew2_ref[e]) + eb2_ref[e]


def _ffn(gid, nblk, xs, ew1, eb1, ew2, eb2, ew3, eb3):
    full = lambda shape: pl.BlockSpec(
        shape, lambda g, gid_ref, nblk_ref: (0,) * len(shape))
    grid_spec = pltpu.PrefetchScalarGridSpec(
        num_scalar_prefetch=2,
        grid=(MAXB,),
        in_specs=[
            pl.BlockSpec(
                (BG, H),
                lambda g, gid_ref, nblk_ref: (
                    jnp.minimum(g, nblk_ref[0] - 1), 0)),
            full((E, I, H)), full((E, 1, I)),
            full((E, H, I)), full((E, 1, H)),
            full((E, I, H)), full((E, 1, I)),
        ],
        out_specs=pl.BlockSpec(
            (BG, H),
            lambda g, gid_ref, nblk_ref: (jnp.minimum(g, nblk_ref[0] - 1), 0)),
    )
    return pl.pallas_call(
        _ffn_block,
        grid_spec=grid_spec,
        out_shape=jax.ShapeDtypeStruct((XS, H), jnp.float32),
    )(gid, nblk, xs, ew1, eb1.reshape(E, 1, I), ew2, eb2.reshape(E, 1, H),
      ew3, eb3.reshape(E, 1, I))


def _combine_body(pos_hbm, ys_hbm, yr_hbm, idx_v, rows_v, sem):
    info = plsc.get_sparse_core_info()
    nc = info.num_cores
    wid = lax.axis_index("s") * nc + lax.axis_index("c")
    n = T // (nc * info.num_subcores)
    base = wid * n
    pltpu.sync_copy(pos_hbm.at[pl.ds(base, n)], idx_v)
    pltpu.async_copy(ys_hbm.at[idx_v], rows_v, sem).wait()
    pltpu.sync_copy(rows_v, yr_hbm.at[pl.ds(base, n)])


def _combine(pos, ys):
    info = plsc.get_sparse_core_info()
    n = T // (info.num_cores * info.num_subcores)
    mesh = plsc.VectorSubcoreMesh(core_axis_name="c", subcore_axis_name="s")
    return pl.kernel(
        _combine_body,
        out_type=jax.ShapeDtypeStruct((T, H), jnp.float32),
        mesh=mesh,
        scratch_types=[
            pltpu.VMEM((n,), jnp.int32),
            pltpu.VMEM((n, H), jnp.float32),
            pltpu.SemaphoreType.DMA,
        ],
        compiler_params=pltpu.CompilerParams(needs_layout_passes=False),
    )(pos, ys)


def _epi_block(yr_ref, wt_ref, z_ref, out_ref):
    out_ref[...] = yr_ref[...] * wt_ref[...] + z_ref[...]


def _epilogue(yr, wt, z):
    return pl.pallas_call(
        _epi_block,
        grid=(NB,),
        in_specs=[pl.BlockSpec((TB, H), lambda b: (b, 0)),
                  pl.BlockSpec((TB, 1), lambda b: (b, 0)),
                  pl.BlockSpec((TB, H), lambda b: (b, 0))],
        out_specs=pl.BlockSpec((TB, H), lambda b: (b, 0)),
        out_shape=jax.ShapeDtypeStruct((T, H), jnp.float32),
    )(yr, wt, z)


@jax.jit
def _moe(x, gate_w, ew1, eb1, ew2, eb2, ew3, eb3, sw1, sb1, sw2, sb2,
         sw3, sb3):
    eid2d, wt2d = _gate(x, gate_w)
    eid = eid2d.reshape(T)
    xs, pos, gid, nblk = _route(eid, x)
    z = _shared(x, sw1, sb1, sw2, sb2, sw3, sb3)
    xs, z = jax.lax.optimization_barrier((xs, z))
    ys = _ffn(gid, nblk, xs, ew1, eb1, ew2, eb2, ew3, eb3)
    yr = _combine(pos, ys)
    return _epilogue(yr, wt2d, z)


def kernel(hidden_states, gate_w, ew1, eb1, ew2, eb2, ew3, eb3, sw1, sb1,
           sw2, sb2, sw3, sb3):
    shape = hidden_states.shape
    x = hidden_states.reshape(-1, H)
    y = _moe(x, gate_w, ew1, eb1, ew2, eb2, ew3, eb3, sw1, sb1, sw2, sb2,
             sw3, sb3)
    return y.reshape(shape)


# speedup vs baseline: 1.1922x; 1.0692x over previous
import functools

import jax
import jax.numpy as jnp
from jax import lax
from jax.experimental import pallas as pl
from jax.experimental.pallas import tpu as pltpu
from jax.experimental.pallas import tpu_sc as plsc

H = 768
I = 128
E = 16
SH_I = 256
T = 2048
TB = 128
NB = T // TB
BG = 128
MAXB = 31
XS = 4096
NT = 16
TPT = T // NT


def _dot_nt(a, b):
    return jax.lax.dot_general(a, b, (((1,), (1,)), ((), ())),
                               preferred_element_type=jnp.float32)


def _gate_block(x_ref, gw_ref, eid_ref, wt_ref):
    xb = x_ref[...]
    scores = jax.lax.dot_general(xb.astype(jnp.bfloat16),
                                 gw_ref[...].astype(jnp.bfloat16),
                                 (((1,), (1,)), ((), ())),
                                 preferred_element_type=jnp.float32)
    smax = jnp.max(scores, axis=-1, keepdims=True)
    p = jnp.exp(scores - smax)
    probs = p / jnp.sum(p, axis=-1, keepdims=True)
    pmax = jnp.max(probs, axis=-1, keepdims=True)
    lane = jax.lax.broadcasted_iota(jnp.int32, (TB, E), 1)
    fm = jnp.min(jnp.where(probs >= pmax, lane, E), axis=-1,
                 keepdims=True)
    eid_ref[...] = fm
    wt_ref[...] = pmax


def _gate(x, gate_w):
    return pl.pallas_call(
        _gate_block,
        grid=(NB,),
        in_specs=[pl.BlockSpec((TB, H), lambda b: (b, 0)),
                  pl.BlockSpec((E, H), lambda b: (0, 0))],
        out_specs=[pl.BlockSpec((TB, 1), lambda b: (b, 0)),
                   pl.BlockSpec((TB, 1), lambda b: (b, 0))],
        out_shape=[jax.ShapeDtypeStruct((T, 1), jnp.int32),
                   jax.ShapeDtypeStruct((T, 1), jnp.float32)],
    )(x, gate_w)


def _lane(vec, i):
    iota = lax.iota(jnp.int32, 16)
    return jnp.sum(jnp.where(iota == i, vec, 0))


def _route_body(eid_hbm, x_hbm, xs_hbm, pos_hbm, gid_hbm, nblk_hbm,
                eid_v, pos_v, rows_v, cnt_v, cnt_all_v, gid_v, nblk_v,
                cnt_sh, sem):
    sid = lax.axis_index("s")
    base = sid * TPT
    iota = lax.iota(jnp.int32, 16)

    rows_dma = pltpu.async_copy(x_hbm.at[pl.ds(base, TPT)], rows_v, sem)
    pltpu.sync_copy(eid_hbm.at[pl.ds(base, TPT)], eid_v)

    cnt = jnp.zeros((16,), jnp.int32)
    chunks = []
    for v in range(TPT // 16):
        ev = eid_v[pl.ds(v * 16, 16)]
        chunks.append(ev)
        for e in range(E):
            c = jnp.sum(jnp.where(ev == e, 1, 0))
            cnt = cnt + jnp.where(iota == e, c, 0)
    cnt_v[...] = cnt

    pltpu.sync_copy(cnt_v, cnt_sh.at[pl.ds(sid * 16, 16)])
    plsc.subcore_barrier()
    pltpu.sync_copy(cnt_sh, cnt_all_v)

    total = jnp.zeros((16,), jnp.int32)
    for w in range(NT):
        total = total + cnt_all_v[pl.ds(w * 16, 16)]
    ptot = jnp.bitwise_and(total + (BG - 1), -BG)
    csum = plsc.cumsum(ptot)
    pbase = csum - ptot

    tbase = pbase
    for w in range(NT):
        before = jnp.full((16,), w, jnp.int32) < sid
        tbase = tbase + jnp.where(before, cnt_all_v[pl.ds(w * 16, 16)], 0)

    ctr = tbase
    for v in range(TPT // 16):
        ev = chunks[v]
        posk = jnp.zeros((16,), jnp.int32)
        for e in range(E):
            m = ev == e
            mi = jnp.where(m, 1, 0)
            r = plsc.cumsum(mi)
            basee = _lane(ctr, e)
            posk = jnp.where(m, basee + r - 1, posk)
            ctr = ctr + jnp.where(iota == e, _lane(r, 15), 0)
        pos_v[pl.ds(v * 16, 16)] = posk

    pltpu.sync_copy(pos_v, pos_hbm.at[pl.ds(base, TPT)])

    rows_dma.wait()
    pltpu.async_copy(rows_v, xs_hbm.at[pos_v], sem).wait()

    @pl.when(sid == 0)
    def _():
        nblk_v[...] = (_lane(csum, 15) >> 7) + jnp.zeros((16,), jnp.int32)
        bstart = pbase >> 7
        for v in range(2):
            pvec = iota + v * 16
            cntb = jnp.zeros((16,), jnp.int32)
            for e in range(E):
                bs = _lane(bstart, e)
                cntb = cntb + jnp.where(bs <= pvec, 1, 0)
            gid_v[pl.ds(v * 16, 16)] = cntb - 1
        pltpu.sync_copy(gid_v, gid_hbm)
        pltpu.sync_copy(nblk_v, nblk_hbm)


def _route(eid, x):
    mesh = plsc.VectorSubcoreMesh(core_axis_name="c", subcore_axis_name="s",
                                  num_cores=1)
    return pl.kernel(
        _route_body,
        out_type=(jax.ShapeDtypeStruct((XS, H), jnp.float32),
                  jax.ShapeDtypeStruct((T,), jnp.int32),
                  jax.ShapeDtypeStruct((32,), jnp.int32),
                  jax.ShapeDtypeStruct((16,), jnp.int32)),
        mesh=mesh,
        scratch_types=[
            pltpu.VMEM((TPT,), jnp.int32),
            pltpu.VMEM((TPT,), jnp.int32),
            pltpu.VMEM((TPT, H), jnp.float32),
            pltpu.VMEM((16,), jnp.int32),
            pltpu.VMEM((NT * 16,), jnp.int32),
            pltpu.VMEM((32,), jnp.int32),
            pltpu.VMEM((16,), jnp.int32),
            pltpu.VMEM_SHARED((NT * 16,), jnp.int32),
            pltpu.SemaphoreType.DMA,
        ],
        compiler_params=pltpu.CompilerParams(needs_layout_passes=False),
    )(eid, x)


def _shared_block(x_ref, sw1_ref, sb1_ref, sw2_ref, sb2_ref, sw3_ref,
                  sb3_ref, z_ref):
    xb = x_ref[...]
    t1 = _dot_nt(xb, sw1_ref[...]) + sb1_ref[...]
    t3 = _dot_nt(xb, sw3_ref[...]) + sb3_ref[...]
    z_ref[...] = _dot_nt(jax.nn.silu(t1) * t3, sw2_ref[...]) + sb2_ref[...]


def _shared(x, sw1, sb1, sw2, sb2, sw3, sb3):
    full = lambda shape: pl.BlockSpec(shape, lambda b: (0,) * len(shape))
    return pl.pallas_call(
        _shared_block,
        grid=(NB,),
        in_specs=[pl.BlockSpec((TB, H), lambda b: (b, 0)),
                  full((SH_I, H)), full((1, SH_I)),
                  full((H, SH_I)), full((1, H)),
                  full((SH_I, H)), full((1, SH_I))],
        out_specs=pl.BlockSpec((TB, H), lambda b: (b, 0)),
        out_shape=jax.ShapeDtypeStruct((T, H), jnp.float32),
    )(x, sw1, sb1.reshape(1, SH_I), sw2, sb2.reshape(1, H), sw3,
      sb3.reshape(1, SH_I))


def _ffn_block(gid_ref, nblk_ref, xs_ref, ew1_ref, eb1_ref, ew2_ref,
               eb2_ref, ew3_ref, eb3_ref, ys_ref):
    g = pl.program_id(0)

    @pl.when(g < nblk_ref[0])
    def _():
        e = gid_ref[g]
        xb = xs_ref[...]
        h1 = _dot_nt(xb, ew1_ref[e]) + eb1_ref[e]
        h3 = _dot_nt(xb, ew3_ref[e]) + eb3_ref[e]
        h = jax.nn.silu(h1) * h3
        ys_ref[...] = _dot_nt(h, ew2_ref[e]) + eb2_ref[e]


def _ffn(gid, nblk, xs, ew1, eb1, ew2, eb2, ew3, eb3):
    full = lambda shape: pl.BlockSpec(
        shape, lambda g, gid_ref, nblk_ref: (0,) * len(shape))
    grid_spec = pltpu.PrefetchScalarGridSpec(
        num_scalar_prefetch=2,
        grid=(MAXB,),
        in_specs=[
            pl.BlockSpec(
                (BG, H),
                lambda g, gid_ref, nblk_ref: (
                    jnp.minimum(g, nblk_ref[0] - 1), 0)),
            full((E, I, H)), full((E, 1, I)),
            full((E, H, I)), full((E, 1, H)),
            full((E, I, H)), full((E, 1, I)),
        ],
        out_specs=pl.BlockSpec(
            (BG, H),
            lambda g, gid_ref, nblk_ref: (jnp.minimum(g, nblk_ref[0] - 1), 0)),
    )
    return pl.pallas_call(
        _ffn_block,
        grid_spec=grid_spec,
        out_shape=jax.ShapeDtypeStruct((XS, H), jnp.float32),
    )(gid, nblk, xs, ew1, eb1.reshape(E, 1, I), ew2, eb2.reshape(E, 1, H),
      ew3, eb3.reshape(E, 1, I))


def _combine_body(pos_hbm, ys_hbm, z_hbm, wt_hbm, out_hbm, idx_v, rows_v,
                  z_v, wt_v, sem, sem2):
    info = plsc.get_sparse_core_info()
    nc = info.num_cores
    wid = lax.axis_index("s") * nc + lax.axis_index("c")
    n = T // (nc * info.num_subcores)
    base = wid * n
    z_dma = pltpu.async_copy(z_hbm.at[pl.ds(base, n)], z_v, sem2)
    pltpu.sync_copy(pos_hbm.at[pl.ds(base, n)], idx_v)
    pltpu.sync_copy(wt_hbm.at[pl.ds(base, n)], wt_v)
    pltpu.async_copy(ys_hbm.at[idx_v], rows_v, sem).wait()
    z_dma.wait()

    zeros16 = jnp.zeros((16,), jnp.int32)

    def row(r, _):
        w = plsc.load_gather(wt_v, [zeros16 + r])
        for c in range(H // 16):
            s = pl.ds(c * 16, 16)
            rows_v[r, s] = rows_v[r, s] * w + z_v[r, s]
        return 0

    lax.fori_loop(0, n, row, 0)
    pltpu.sync_copy(rows_v, out_hbm.at[pl.ds(base, n)])


def _combine(pos, ys, z, wt):
    info = plsc.get_sparse_core_info()
    n = T // (info.num_cores * info.num_subcores)
    mesh = plsc.VectorSubcoreMesh(core_axis_name="c", subcore_axis_name="s")
    return pl.kernel(
        _combine_body,
        out_type=jax.ShapeDtypeStruct((T, H), jnp.float32),
        mesh=mesh,
        scratch_types=[
            pltpu.VMEM((n,), jnp.int32),
            pltpu.VMEM((n, H), jnp.float32),
            pltpu.VMEM((n, H), jnp.float32),
            pltpu.VMEM((n,), jnp.float32),
            pltpu.SemaphoreType.DMA,
            pltpu.SemaphoreType.DMA,
        ],
        compiler_params=pltpu.CompilerParams(needs_layout_passes=False),
    )(pos, ys, z, wt)


def _epi_block(yr_ref, wt_ref, z_ref, out_ref):
    out_ref[...] = yr_ref[...] * wt_ref[...] + z_ref[...]


def _epilogue(yr, wt, z):
    return pl.pallas_call(
        _epi_block,
        grid=(NB,),
        in_specs=[pl.BlockSpec((TB, H), lambda b: (b, 0)),
                  pl.BlockSpec((TB, 1), lambda b: (b, 0)),
                  pl.BlockSpec((TB, H), lambda b: (b, 0))],
        out_specs=pl.BlockSpec((TB, H), lambda b: (b, 0)),
        out_shape=jax.ShapeDtypeStruct((T, H), jnp.float32),
    )(yr, wt, z)


@jax.jit
def _moe(x, gate_w, ew1, eb1, ew2, eb2, ew3, eb3, sw1, sb1, sw2, sb2,
         sw3, sb3):
    eid2d, wt2d = _gate(x, gate_w)
    eid = eid2d.reshape(T)
    xs, pos, gid, nblk = _route(eid, x)
    z = _shared(x, sw1, sb1, sw2, sb2, sw3, sb3)
    xs, z = jax.lax.optimization_barrier((xs, z))
    ys = _ffn(gid, nblk, xs, ew1, eb1, ew2, eb2, ew3, eb3)
    return _combine(pos, ys, z, wt2d.reshape(T))


def kernel(hidden_states, gate_w, ew1, eb1, ew2, eb2, ew3, eb3, sw1, sb1,
           sw2, sb2, sw3, sb3):
    shape = hidden_states.shape
    x = hidden_states.reshape(-1, H)
    y = _moe(x, gate_w, ew1, eb1, ew2, eb2, ew3, eb3, sw1, sb1, sw2, sb2,
             sw3, sb3)
    return y.reshape(shape)


# speedup vs baseline: 1.3165x; 1.1043x over previous
import functools

import jax
import jax.numpy as jnp
from jax import lax
from jax.experimental import pallas as pl
from jax.experimental.pallas import tpu as pltpu
from jax.experimental.pallas import tpu_sc as plsc

H = 768
I = 128
E = 16
SH_I = 256
T = 2048
TB = 128
NB = T // TB
BG = 128
MAXB = 31
XS = 4096
NT = 16
TPT = T // NT


def _dot_nt(a, b):
    return jax.lax.dot_general(a, b, (((1,), (1,)), ((), ())),
                               preferred_element_type=jnp.float32)


def _gate_block(x_ref, gw_ref, eid_ref, wt_ref):
    xb = x_ref[...]
    scores = jax.lax.dot_general(xb.astype(jnp.bfloat16),
                                 gw_ref[...].astype(jnp.bfloat16),
                                 (((1,), (1,)), ((), ())),
                                 preferred_element_type=jnp.float32)
    smax = jnp.max(scores, axis=-1, keepdims=True)
    p = jnp.exp(scores - smax)
    probs = p / jnp.sum(p, axis=-1, keepdims=True)
    pmax = jnp.max(probs, axis=-1, keepdims=True)
    lane = jax.lax.broadcasted_iota(jnp.int32, scores.shape, 1)
    fm = jnp.min(jnp.where(probs >= pmax, lane, E), axis=-1,
                 keepdims=True)
    eid_ref[...] = fm
    wt_ref[...] = pmax


def _gate(x, gate_w):
    tba = 512
    return pl.pallas_call(
        _gate_block,
        grid=(T // tba,),
        in_specs=[pl.BlockSpec((tba, H), lambda b: (b, 0)),
                  pl.BlockSpec((E, H), lambda b: (0, 0))],
        out_specs=[pl.BlockSpec((tba, 1), lambda b: (b, 0)),
                   pl.BlockSpec((tba, 1), lambda b: (b, 0))],
        out_shape=[jax.ShapeDtypeStruct((T, 1), jnp.int32),
                   jax.ShapeDtypeStruct((T, 1), jnp.float32)],
    )(x, gate_w)


def _lane(vec, i):
    iota = lax.iota(jnp.int32, 16)
    return jnp.sum(jnp.where(iota == i, vec, 0))


def _route_body(eid_hbm, x_hbm, xs_hbm, pos_hbm, gid_hbm, nblk_hbm,
                eid_v, pos_v, rows_v, cnt_v, cnt_all_v, gid_v, nblk_v,
                cnt_sh, sem):
    sid = lax.axis_index("s")
    base = sid * TPT
    iota = lax.iota(jnp.int32, 16)

    rows_dma = pltpu.async_copy(x_hbm.at[pl.ds(base, TPT)], rows_v, sem)
    pltpu.sync_copy(eid_hbm.at[pl.ds(base, TPT)], eid_v)

    cnt = jnp.zeros((16,), jnp.int32)
    chunks = []
    for v in range(TPT // 16):
        ev = eid_v[pl.ds(v * 16, 16)]
        chunks.append(ev)
        for e in range(E):
            c = jnp.sum(jnp.where(ev == e, 1, 0))
            cnt = cnt + jnp.where(iota == e, c, 0)
    cnt_v[...] = cnt

    pltpu.sync_copy(cnt_v, cnt_sh.at[pl.ds(sid * 16, 16)])
    plsc.subcore_barrier()
    pltpu.sync_copy(cnt_sh, cnt_all_v)

    total = jnp.zeros((16,), jnp.int32)
    for w in range(NT):
        total = total + cnt_all_v[pl.ds(w * 16, 16)]
    ptot = jnp.bitwise_and(total + (BG - 1), -BG)
    csum = plsc.cumsum(ptot)
    pbase = csum - ptot

    tbase = pbase
    for w in range(NT):
        before = jnp.full((16,), w, jnp.int32) < sid
        tbase = tbase + jnp.where(before, cnt_all_v[pl.ds(w * 16, 16)], 0)

    ctr = tbase
    for v in range(TPT // 16):
        ev = chunks[v]
        posk = jnp.zeros((16,), jnp.int32)
        for e in range(E):
            m = ev == e
            mi = jnp.where(m, 1, 0)
            r = plsc.cumsum(mi)
            basee = _lane(ctr, e)
            posk = jnp.where(m, basee + r - 1, posk)
            ctr = ctr + jnp.where(iota == e, _lane(r, 15), 0)
        pos_v[pl.ds(v * 16, 16)] = posk

    pltpu.sync_copy(pos_v, pos_hbm.at[pl.ds(base, TPT)])

    rows_dma.wait()
    pltpu.async_copy(rows_v, xs_hbm.at[pos_v], sem).wait()

    @pl.when(sid == 0)
    def _():
        nblk_v[...] = (_lane(csum, 15) >> 7) + jnp.zeros((16,), jnp.int32)
        bstart = pbase >> 7
        for v in range(2):
            pvec = iota + v * 16
            cntb = jnp.zeros((16,), jnp.int32)
            for e in range(E):
                bs = _lane(bstart, e)
                cntb = cntb + jnp.where(bs <= pvec, 1, 0)
            gid_v[pl.ds(v * 16, 16)] = cntb - 1
        pltpu.sync_copy(gid_v, gid_hbm)
        pltpu.sync_copy(nblk_v, nblk_hbm)


def _route(eid, x):
    mesh = plsc.VectorSubcoreMesh(core_axis_name="c", subcore_axis_name="s",
                                  num_cores=1)
    return pl.kernel(
        _route_body,
        out_type=(jax.ShapeDtypeStruct((XS, H), jnp.float32),
                  jax.ShapeDtypeStruct((T,), jnp.int32),
                  jax.ShapeDtypeStruct((32,), jnp.int32),
                  jax.ShapeDtypeStruct((16,), jnp.int32)),
        mesh=mesh,
        scratch_types=[
            pltpu.VMEM((TPT,), jnp.int32),
            pltpu.VMEM((TPT,), jnp.int32),
            pltpu.VMEM((TPT, H), jnp.float32),
            pltpu.VMEM((16,), jnp.int32),
            pltpu.VMEM((NT * 16,), jnp.int32),
            pltpu.VMEM((32,), jnp.int32),
            pltpu.VMEM((16,), jnp.int32),
            pltpu.VMEM_SHARED((NT * 16,), jnp.int32),
            pltpu.SemaphoreType.DMA,
        ],
        compiler_params=pltpu.CompilerParams(needs_layout_passes=False),
    )(eid, x)


def _shared_block(x_ref, sw1_ref, sb1_ref, sw2_ref, sb2_ref, sw3_ref,
                  sb3_ref, z_ref):
    xb = x_ref[...]
    t1 = _dot_nt(xb, sw1_ref[...]) + sb1_ref[...]
    t3 = _dot_nt(xb, sw3_ref[...]) + sb3_ref[...]
    z_ref[...] = _dot_nt(jax.nn.silu(t1) * t3, sw2_ref[...]) + sb2_ref[...]


def _shared(x, sw1, sb1, sw2, sb2, sw3, sb3):
    full = lambda shape: pl.BlockSpec(shape, lambda b: (0,) * len(shape))
    return pl.pallas_call(
        _shared_block,
        grid=(NB,),
        in_specs=[pl.BlockSpec((TB, H), lambda b: (b, 0)),
                  full((SH_I, H)), full((1, SH_I)),
                  full((H, SH_I)), full((1, H)),
                  full((SH_I, H)), full((1, SH_I))],
        out_specs=pl.BlockSpec((TB, H), lambda b: (b, 0)),
        out_shape=jax.ShapeDtypeStruct((T, H), jnp.float32),
    )(x, sw1, sb1.reshape(1, SH_I), sw2, sb2.reshape(1, H), sw3,
      sb3.reshape(1, SH_I))


def _ffn_block(gid_ref, nblk_ref, xs_ref, ew1_ref, eb1_ref, ew2_ref,
               eb2_ref, ew3_ref, eb3_ref, ys_ref):
    g = pl.program_id(0)

    @pl.when(g < nblk_ref[0])
    def _():
        e = gid_ref[g]
        xb = xs_ref[...].astype(jnp.bfloat16)
        h1 = _dot_nt(xb, ew1_ref[e].astype(jnp.bfloat16)) + eb1_ref[e]
        h3 = _dot_nt(xb, ew3_ref[e].astype(jnp.bfloat16)) + eb3_ref[e]
        h = (jax.nn.silu(h1) * h3).astype(jnp.bfloat16)
        ys_ref[...] = _dot_nt(h, ew2_ref[e].astype(jnp.bfloat16)) + eb2_ref[e]


def _ffn(gid, nblk, xs, ew1, eb1, ew2, eb2, ew3, eb3):
    full = lambda shape: pl.BlockSpec(
        shape, lambda g, gid_ref, nblk_ref: (0,) * len(shape))
    grid_spec = pltpu.PrefetchScalarGridSpec(
        num_scalar_prefetch=2,
        grid=(MAXB,),
        in_specs=[
            pl.BlockSpec(
                (BG, H),
                lambda g, gid_ref, nblk_ref: (
                    jnp.minimum(g, nblk_ref[0] - 1), 0)),
            full((E, I, H)), full((E, 1, I)),
            full((E, H, I)), full((E, 1, H)),
            full((E, I, H)), full((E, 1, I)),
        ],
        out_specs=pl.BlockSpec(
            (BG, H),
            lambda g, gid_ref, nblk_ref: (jnp.minimum(g, nblk_ref[0] - 1), 0)),
    )
    return pl.pallas_call(
        _ffn_block,
        grid_spec=grid_spec,
        out_shape=jax.ShapeDtypeStruct((XS, H), jnp.float32),
    )(gid, nblk, xs, ew1, eb1.reshape(E, 1, I), ew2, eb2.reshape(E, 1, H),
      ew3, eb3.reshape(E, 1, I))


def _combine_body(pos_hbm, ys_hbm, z_hbm, wt_hbm, out_hbm, idx_v, rows_v,
                  z_v, wt_v, sem, sem2):
    info = plsc.get_sparse_core_info()
    nc = info.num_cores
    wid = lax.axis_index("s") * nc + lax.axis_index("c")
    n = T // (nc * info.num_subcores)
    base = wid * n
    z_dma = pltpu.async_copy(z_hbm.at[pl.ds(base, n)], z_v, sem2)
    pltpu.sync_copy(pos_hbm.at[pl.ds(base, n)], idx_v)
    pltpu.sync_copy(wt_hbm.at[pl.ds(base, n)], wt_v)
    pltpu.async_copy(ys_hbm.at[idx_v], rows_v, sem).wait()
    z_dma.wait()

    zeros16 = jnp.zeros((16,), jnp.int32)

    def row(r, _):
        w = plsc.load_gather(wt_v, [zeros16 + r])
        for c in range(H // 16):
            s = pl.ds(c * 16, 16)
            rows_v[r, s] = rows_v[r, s] * w + z_v[r, s]
        return 0

    lax.fori_loop(0, n, row, 0)
    pltpu.sync_copy(rows_v, out_hbm.at[pl.ds(base, n)])


def _combine(pos, ys, z, wt):
    info = plsc.get_sparse_core_info()
    n = T // (info.num_cores * info.num_subcores)
    mesh = plsc.VectorSubcoreMesh(core_axis_name="c", subcore_axis_name="s")
    return pl.kernel(
        _combine_body,
        out_type=jax.ShapeDtypeStruct((T, H), jnp.float32),
        mesh=mesh,
        scratch_types=[
            pltpu.VMEM((n,), jnp.int32),
            pltpu.VMEM((n, H), jnp.float32),
            pltpu.VMEM((n, H), jnp.float32),
            pltpu.VMEM((n,), jnp.float32),
            pltpu.SemaphoreType.DMA,
            pltpu.SemaphoreType.DMA,
        ],
        compiler_params=pltpu.CompilerParams(needs_layout_passes=False),
    )(pos, ys, z, wt)


def _epi_block(yr_ref, wt_ref, z_ref, out_ref):
    out_ref[...] = yr_ref[...] * wt_ref[...] + z_ref[...]


def _epilogue(yr, wt, z):
    return pl.pallas_call(
        _epi_block,
        grid=(NB,),
        in_specs=[pl.BlockSpec((TB, H), lambda b: (b, 0)),
                  pl.BlockSpec((TB, 1), lambda b: (b, 0)),
                  pl.BlockSpec((TB, H), lambda b: (b, 0))],
        out_specs=pl.BlockSpec((TB, H), lambda b: (b, 0)),
        out_shape=jax.ShapeDtypeStruct((T, H), jnp.float32),
    )(yr, wt, z)


@jax.jit
def _moe(x, gate_w, ew1, eb1, ew2, eb2, ew3, eb3, sw1, sb1, sw2, sb2,
         sw3, sb3):
    eid2d, wt2d = _gate(x, gate_w)
    eid = eid2d.reshape(T)
    xs, pos, gid, nblk = _route(eid, x)
    z = _shared(x, sw1, sb1, sw2, sb2, sw3, sb3)
    xs, z = jax.lax.optimization_barrier((xs, z))
    ys = _ffn(gid, nblk, xs, ew1, eb1, ew2, eb2, ew3, eb3)
    return _combine(pos, ys, z, wt2d.reshape(T))


def kernel(hidden_states, gate_w, ew1, eb1, ew2, eb2, ew3, eb3, sw1, sb1,
           sw2, sb2, sw3, sb3):
    shape = hidden_states.shape
    x = hidden_states.reshape(-1, H)
    y = _moe(x, gate_w, ew1, eb1, ew2, eb2, ew3, eb3, sw1, sb1, sw2, sb2,
             sw3, sb3)
    return y.reshape(shape)
